# baseline (speedup 1.0000x reference)
import functools

import jax
import jax.numpy as jnp
from jax import lax
from jax.experimental import pallas as pl
from jax.experimental.pallas import tpu as pltpu
from jax.experimental.pallas import tpu_sc as plsc

N_NODES = 10000
N_EDGES = 320000
D = 128
H = 8
HD = 16
NP = 10112
NB = NP // 128
ACCW = 144
NWORK = 32
EB = 80
NBATCH = 128
EPW = NBATCH * EB
E_PAD = NWORK * EPW
ROWS_PER_TILE = NP // 16
F32 = jnp.float32
_PREC = lax.Precision.HIGHEST


def _tc1_body(x_ref, w_ref, acat_ref, h_ref, asd_ref):
    h = jnp.dot(x_ref[...], w_ref[...], precision=_PREC)
    h_ref[...] = h
    asd_ref[...] = jnp.dot(h, acat_ref[...], precision=_PREC)


def _normalize(accs, h, asd, eh, scale, shift):
    a = accs[0] + accs[1]
    num = a[:, :D]
    den = a[:, D:D + H]
    al = asd[:, :H] + asd[:, H:2 * H]
    w_self = jnp.exp(jnp.maximum(al, 0.2 * al))
    wexp = jnp.dot(w_self, eh, precision=_PREC)
    dexp = jnp.dot(den + w_self, eh, precision=_PREC)
    y = (num + wexp * h) / dexp
    y = y * scale + shift
    return jnp.where(y > 0, y, jnp.exp(y) - 1.0)


def _tc_mid_body(accs_ref, h_ref, asd_ref, w_ref, acat_ref, eh_ref, sc_ref,
                 sh_ref, h2_ref, asd2_ref):
    y = _normalize(accs_ref[...], h_ref[...], asd_ref[...], eh_ref[...],
                   sc_ref[...], sh_ref[...])
    h2 = jnp.dot(y, w_ref[...], precision=_PREC)
    h2_ref[...] = h2
    asd2_ref[...] = jnp.dot(h2, acat_ref[...], precision=_PREC)


def _tc_last_body(accs_ref, h_ref, asd_ref, w_ref, eh_ref, sc_ref, sh_ref,
                  h3_ref):
    y = _normalize(accs_ref[...], h_ref[...], asd_ref[...], eh_ref[...],
                   sc_ref[...], sh_ref[...])
    h3_ref[...] = jnp.dot(y, w_ref[...], precision=_PREC)


def _tc4_body(num0_ref, num1_ref, den0_ref, den1_ref, h_ref, al_ref, b3_ref,
              out_ref):
    num = num0_ref[...] + num1_ref[...]
    den = den0_ref[...] + den1_ref[...]
    al = al_ref[...]
    w_self = jnp.exp(jnp.maximum(al, 0.2 * al))
    r = (num + w_self * h_ref[...]) / (den + w_self) + b3_ref[...]
    out_ref[...] = 1.0 / (1.0 + jnp.exp(-r))


def _tc1(xp, W1, acat1):
    return pl.pallas_call(
        _tc1_body,
        grid=(NB,),
        in_specs=[
            pl.BlockSpec((128, D), lambda i: (i, 0)),
            pl.BlockSpec((D, D), lambda i: (0, 0)),
            pl.BlockSpec((D, 2 * H), lambda i: (0, 0)),
        ],
        out_specs=[
            pl.BlockSpec((128, D), lambda i: (i, 0)),
            pl.BlockSpec((128, 2 * H), lambda i: (i, 0)),
        ],
        out_shape=[
            jax.ShapeDtypeStruct((NP, D), F32),
            jax.ShapeDtypeStruct((NP, 2 * H), F32),
        ],
    )(xp, W1, acat1)


def _tc_mid(accs, h, asd, W, acat, eh, scale, shift):
    return pl.pallas_call(
        _tc_mid_body,
        grid=(NB,),
        in_specs=[
            pl.BlockSpec((2, 128, ACCW), lambda i: (0, i, 0)),
            pl.BlockSpec((128, D), lambda i: (i, 0)),
            pl.BlockSpec((128, 2 * H), lambda i: (i, 0)),
            pl.BlockSpec((D, D), lambda i: (0, 0)),
            pl.BlockSpec((D, 2 * H), lambda i: (0, 0)),
            pl.BlockSpec((H, D), lambda i: (0, 0)),
            pl.BlockSpec((1, D), lambda i: (0, 0)),
            pl.BlockSpec((1, D), lambda i: (0, 0)),
        ],
        out_specs=[
            pl.BlockSpec((128, D), lambda i: (i, 0)),
            pl.BlockSpec((128, 2 * H), lambda i: (i, 0)),
        ],
        out_shape=[
            jax.ShapeDtypeStruct((NP, D), F32),
            jax.ShapeDtypeStruct((NP, 2 * H), F32),
        ],
    )(accs, h, asd, W, acat, eh, scale, shift)


def _tc_last(accs, h, asd, W3p, eh, scale, shift):
    return pl.pallas_call(
        _tc_last_body,
        grid=(NB,),
        in_specs=[
            pl.BlockSpec((2, 128, ACCW), lambda i: (0, i, 0)),
            pl.BlockSpec((128, D), lambda i: (i, 0)),
            pl.BlockSpec((128, 2 * H), lambda i: (i, 0)),
            pl.BlockSpec((D, D), lambda i: (0, 0)),
            pl.BlockSpec((H, D), lambda i: (0, 0)),
            pl.BlockSpec((1, D), lambda i: (0, 0)),
            pl.BlockSpec((1, D), lambda i: (0, 0)),
        ],
        out_specs=pl.BlockSpec((128, D), lambda i: (i, 0)),
        out_shape=jax.ShapeDtypeStruct((NP, D), F32),
    )(accs, h, asd, W3p, eh, scale, shift)


def _tc4(num0, num1, den0, den1, h2d, al2d, b3s):
    return pl.pallas_call(
        _tc4_body,
        out_shape=jax.ShapeDtypeStruct((NB, 128), F32),
    )(num0, num1, den0, den1, h2d, al2d, b3s)


def _vperm(x, idx):
    return lax.gather(
        x, idx[:, None],
        lax.GatherDimensionNumbers(offset_dims=(), collapsed_slice_dims=(0,),
                                   start_index_map=(0,)),
        (1,), mode=lax.GatherScatterMode.PROMISE_IN_BOUNDS)


def _sc_edge_body(src_hbm, dst_hbm, asd_hbm, h_hbm, zeros_hbm, out_hbm,
                  idx_s2, idx_d2, asd_s2, asd_d2, hrows2, contrib,
                  acc_sh, semg0, semg1, semi0, semi1):
    c = lax.axis_index("c")
    s = lax.axis_index("s")
    wid = s * 2 + c
    r0 = s * ROWS_PER_TILE
    pltpu.sync_copy(zeros_hbm.at[pl.ds(r0, ROWS_PER_TILE)],
                    acc_sh.at[pl.ds(r0, ROWS_PER_TILE)])
    plsc.subcore_barrier()

    shift_idx = (lax.iota(jnp.int32, 16) + 8) & 15
    splat = [jnp.full((16,), hh, jnp.int32) for hh in range(H)]
    semg = (semg0, semg1)
    semi = (semi0, semi1)

    def issue(gb, sl):
        return (pltpu.async_copy(asd_hbm.at[idx_s2.at[sl]], asd_s2.at[gb],
                                 semg[gb]),
                pltpu.async_copy(asd_hbm.at[idx_d2.at[sl]], asd_d2.at[gb],
                                 semg[gb]),
                pltpu.async_copy(h_hbm.at[idx_s2.at[sl]], hrows2.at[gb],
                                 semg[gb]))

    def drain(cps):
        for cp in cps:
            cp.wait()

    for b in range(2):
        pltpu.sync_copy(src_hbm.at[wid, b], idx_s2.at[b])
        pltpu.sync_copy(dst_hbm.at[wid, b], idx_d2.at[b])
    gather_spec = [issue(0, 0), issue(1, 1)]

    def outer_body(io, carry):
        for b in range(4):
            i = io * 4 + b
            gb = b % 2
            sl = (b + 2) % 4
            drain(gather_spec[gb])
            inext = lax.rem(i + 2, NBATCH)
            cpi = (pltpu.async_copy(src_hbm.at[wid, inext], idx_s2.at[sl],
                                    semi[0]),
                   pltpu.async_copy(dst_hbm.at[wid, inext], idx_d2.at[sl],
                                    semi[0]))

            def edge_body(e, carry2):
                rs = asd_s2[gb, e, :]
                rd = asd_d2[gb, e, :]
                rd8 = _vperm(rd, shift_idx)
                alpha = rs + rd8
                w = jnp.exp(jnp.maximum(alpha, 0.2 * alpha))
                contrib[e, pl.ds(D, 16)] = w
                for hh in range(H):
                    wv = _vperm(w, splat[hh])
                    contrib[e, pl.ds(hh * 16, 16)] = (
                        wv * hrows2[gb, e, pl.ds(hh * 16, 16)])
                return carry2

            lax.fori_loop(0, EB, edge_body, 0)
            pltpu.sync_copy(contrib, acc_sh.at[idx_d2.at[b]],
                            add=True)
            drain(cpi)
            issue(gb, sl)
        return carry

    lax.fori_loop(0, NBATCH // 4, outer_body, 0)
    drain(gather_spec[0])
    drain(gather_spec[1])
    plsc.subcore_barrier()
    pltpu.sync_copy(acc_sh.at[pl.ds(r0, ROWS_PER_TILE)],
                    out_hbm.at[c, pl.ds(r0, ROWS_PER_TILE)])


def _sc_edge3_body(src_hbm, dst_hbm, h3_hbm, att_hbm, zeros_hbm, out_hbm,
                   idx_s, idx_d, h3_v, att_v, contrib, acc_sh, sem):
    c = lax.axis_index("c")
    s = lax.axis_index("s")
    wid = s * 2 + c
    r0 = s * ROWS_PER_TILE
    pltpu.sync_copy(zeros_hbm.at[pl.ds(r0, ROWS_PER_TILE)],
                    acc_sh.at[pl.ds(r0, ROWS_PER_TILE)])
    pltpu.sync_copy(h3_hbm, h3_v)
    pltpu.sync_copy(att_hbm, att_v)
    plsc.subcore_barrier()

    sv = att_v[0, :]
    dv = att_v[1, :]
    zcol = jnp.zeros((16,), jnp.int32)
    ocol = jnp.full((16,), 1, jnp.int32)
    zero16 = jnp.zeros((16,), F32)

    def zrow(r, carry):
        contrib[r, :] = zero16
        return carry

    lax.fori_loop(0, EB, zrow, 0)

    def batch_body(i, carry):
        pltpu.sync_copy(src_hbm.at[wid, i], idx_s)
        pltpu.sync_copy(dst_hbm.at[wid, i], idx_d)
        for g in range(EB // 16):
            src16 = idx_s[pl.ds(g * 16, 16)]
            dst16 = idx_d[pl.ds(g * 16, 16)]
            hsrc = plsc.load_gather(h3_v, [src16])
            hdst = plsc.load_gather(h3_v, [dst16])
            al = sv * hsrc + dv * hdst
            w = jnp.exp(jnp.maximum(al, 0.2 * al))
            eid = lax.iota(jnp.int32, 16) + g * 16
            plsc.store_scatter(contrib, [eid, zcol], w * hsrc)
            plsc.store_scatter(contrib, [eid, ocol], w)
        pltpu.sync_copy(contrib, acc_sh.at[idx_d], add=True)
        return carry

    lax.fori_loop(0, NBATCH, batch_body, 0)
    plsc.subcore_barrier()
    pltpu.sync_copy(acc_sh.at[pl.ds(r0, ROWS_PER_TILE)],
                    out_hbm.at[c, pl.ds(r0, ROWS_PER_TILE)])


@functools.lru_cache(maxsize=None)
def _sc_kernels():
    mesh = plsc.VectorSubcoreMesh(core_axis_name="c", subcore_axis_name="s",
                                  num_cores=2, num_subcores=16)
    params = pltpu.CompilerParams(use_tc_tiling_on_sc=False,
                                  needs_layout_passes=False)
    sc_edge = pl.kernel(
        _sc_edge_body,
        out_type=jax.ShapeDtypeStruct((2, NP, ACCW), F32),
        mesh=mesh,
        compiler_params=params,
        scratch_types=[
            pltpu.VMEM((4, EB), jnp.int32),
            pltpu.VMEM((4, EB), jnp.int32),
            pltpu.VMEM((2, EB, 16), F32),
            pltpu.VMEM((2, EB, 16), F32),
            pltpu.VMEM((2, EB, D), F32),
            pltpu.VMEM((EB, ACCW), F32),
            pltpu.VMEM_SHARED((NP, ACCW), F32),
            pltpu.SemaphoreType.DMA,
            pltpu.SemaphoreType.DMA,
            pltpu.SemaphoreType.DMA,
            pltpu.SemaphoreType.DMA,
        ],
    )
    sc_edge3 = pl.kernel(
        _sc_edge3_body,
        out_type=jax.ShapeDtypeStruct((2, NP, 16), F32),
        mesh=mesh,
        compiler_params=params,
        scratch_types=[
            pltpu.VMEM((EB,), jnp.int32),
            pltpu.VMEM((EB,), jnp.int32),
            pltpu.VMEM((NP,), F32),
            pltpu.VMEM((2, 16), F32),
            pltpu.VMEM((EB, 16), F32),
            pltpu.VMEM_SHARED((NP, 16), F32),
            pltpu.SemaphoreType.DMA,
        ],
    )
    return sc_edge, sc_edge3


def _acat(asrc, adst):
    eye = jnp.eye(H, dtype=F32)
    a_s = (asrc[:, :, None] * eye[:, None, :]).reshape(H * HD, H)
    a_d = (adst[:, :, None] * eye[:, None, :]).reshape(H * HD, H)
    return jnp.concatenate([a_s, a_d], axis=1)


def kernel(x, edge_index, W1, a_src1, a_dst1, b1, g1, be1,
           W2, a_src2, a_dst2, b2, g2, be2,
           W3, a_src3, a_dst3, b3):
    xp = jnp.zeros((NP, D), F32).at[:N_NODES].set(x)
    src = jnp.zeros((E_PAD,), jnp.int32).at[:N_EDGES].set(
        edge_index[0]).reshape(NWORK, NBATCH, EB)
    dst = jnp.full((E_PAD,), NP - 1, jnp.int32).at[:N_EDGES].set(
        edge_index[1]).reshape(NWORK, NBATCH, EB)

    acat1 = _acat(a_src1, a_dst1)
    acat2 = _acat(a_src2, a_dst2)
    eh = (jnp.eye(H, dtype=F32)[:, :, None]
          * jnp.ones((HD,), F32)).reshape(H, H * HD)
    inv = 1.0 / jnp.sqrt(jnp.float32(1.0 + 1e-5))
    sc1 = (g1 * inv).reshape(1, D)
    sh1 = (b1 * g1 * inv + be1).reshape(1, D)
    sc2 = (g2 * inv).reshape(1, D)
    sh2 = (b2 * g2 * inv + be2).reshape(1, D)

    s3 = a_src3[0, 0]
    d3 = a_dst3[0, 0]
    W3p = jnp.zeros((D, D), F32)
    W3p = W3p.at[:, 0].set(W3[:, 0])
    W3p = W3p.at[:, 1].set(W3[:, 0] * (s3 + d3))
    att3 = jnp.stack([jnp.full((16,), s3, F32), jnp.full((16,), d3, F32)])

    zeros_acc = jnp.zeros((NP, ACCW), F32)
    zeros_a3 = jnp.zeros((NP, 16), F32)
    _sc_edge, _sc_edge3 = _sc_kernels()

    h1, asd1 = _tc1(xp, W1, acat1)
    accs1 = _sc_edge(src, dst, asd1, h1, zeros_acc)
    h2, asd2 = _tc_mid(accs1, h1, asd1, W2, acat2, eh, sc1, sh1)
    accs2 = _sc_edge(src, dst, asd2, h2, zeros_acc)
    h3full = _tc_last(accs2, h2, asd2, W3p, eh, sc2, sh2)

    h3 = h3full[:, 0]
    al3 = h3full[:, 1]
    accs3 = _sc_edge3(src, dst, h3, att3, zeros_a3)

    num0 = accs3[0, :, 0].reshape(NB, 128)
    num1 = accs3[1, :, 0].reshape(NB, 128)
    den0 = accs3[0, :, 1].reshape(NB, 128)
    den1 = accs3[1, :, 1].reshape(NB, 128)
    h2d = h3.reshape(NB, 128)
    al2d = al3.reshape(NB, 128)
    b3s = jnp.broadcast_to(b3.reshape(1, 1), (1, 128)).astype(F32)

    out2d = _tc4(num0, num1, den0, den1, h2d, al2d, b3s)
    return out2d.reshape(NP, 1)[:N_NODES]


# speedup vs baseline: 1.1488x; 1.1488x over previous
import functools

import jax
import jax.numpy as jnp
from jax import lax
from jax.experimental import pallas as pl
from jax.experimental.pallas import tpu as pltpu
from jax.experimental.pallas import tpu_sc as plsc

N_NODES = 10000
N_EDGES = 320000
D = 128
H = 8
HD = 16
NP = 10112
NB = NP // 128
ACCW = 144
NWORK = 32
EB = 80
NBATCH = 128
EPW = NBATCH * EB
E_PAD = NWORK * EPW
ROWS_PER_TILE = NP // 16
F32 = jnp.float32
_PREC = lax.Precision.HIGHEST


def _tc1_body(x_ref, w_ref, acat_ref, h_ref, asd_ref):
    h = jnp.dot(x_ref[...], w_ref[...], precision=_PREC)
    h_ref[...] = h
    asd_ref[...] = jnp.dot(h, acat_ref[...], precision=_PREC)


def _normalize(accs, h, asd, eh, scale, shift):
    a = accs[0] + accs[1]
    num = a[:, :D]
    den = a[:, D:D + H]
    al = asd[:, :H] + asd[:, H:2 * H]
    w_self = jnp.exp(jnp.maximum(al, 0.2 * al))
    wexp = jnp.dot(w_self, eh, precision=_PREC)
    dexp = jnp.dot(den + w_self, eh, precision=_PREC)
    y = (num + wexp * h) / dexp
    y = y * scale + shift
    return jnp.where(y > 0, y, jnp.exp(y) - 1.0)


def _tc_mid_body(accs_ref, h_ref, asd_ref, w_ref, acat_ref, eh_ref, sc_ref,
                 sh_ref, h2_ref, asd2_ref):
    y = _normalize(accs_ref[...], h_ref[...], asd_ref[...], eh_ref[...],
                   sc_ref[...], sh_ref[...])
    h2 = jnp.dot(y, w_ref[...], precision=_PREC)
    h2_ref[...] = h2
    asd2_ref[...] = jnp.dot(h2, acat_ref[...], precision=_PREC)


def _tc_last_body(accs_ref, h_ref, asd_ref, w_ref, eh_ref, sc_ref, sh_ref,
                  h3_ref):
    y = _normalize(accs_ref[...], h_ref[...], asd_ref[...], eh_ref[...],
                   sc_ref[...], sh_ref[...])
    h3_ref[...] = jnp.dot(y, w_ref[...], precision=_PREC)


def _tc4_body(num0_ref, num1_ref, den0_ref, den1_ref, h_ref, al_ref, b3_ref,
              out_ref):
    num = num0_ref[...] + num1_ref[...]
    den = den0_ref[...] + den1_ref[...]
    al = al_ref[...]
    w_self = jnp.exp(jnp.maximum(al, 0.2 * al))
    r = (num + w_self * h_ref[...]) / (den + w_self) + b3_ref[...]
    out_ref[...] = 1.0 / (1.0 + jnp.exp(-r))


def _tc1(xp, W1, acat1):
    return pl.pallas_call(
        _tc1_body,
        grid=(NB,),
        in_specs=[
            pl.BlockSpec((128, D), lambda i: (i, 0)),
            pl.BlockSpec((D, D), lambda i: (0, 0)),
            pl.BlockSpec((D, 2 * H), lambda i: (0, 0)),
        ],
        out_specs=[
            pl.BlockSpec((128, D), lambda i: (i, 0)),
            pl.BlockSpec((128, 2 * H), lambda i: (i, 0)),
        ],
        out_shape=[
            jax.ShapeDtypeStruct((NP, D), F32),
            jax.ShapeDtypeStruct((NP, 2 * H), F32),
        ],
    )(xp, W1, acat1)


def _tc_mid(accs, h, asd, W, acat, eh, scale, shift):
    return pl.pallas_call(
        _tc_mid_body,
        grid=(NB,),
        in_specs=[
            pl.BlockSpec((2, 128, ACCW), lambda i: (0, i, 0)),
            pl.BlockSpec((128, D), lambda i: (i, 0)),
            pl.BlockSpec((128, 2 * H), lambda i: (i, 0)),
            pl.BlockSpec((D, D), lambda i: (0, 0)),
            pl.BlockSpec((D, 2 * H), lambda i: (0, 0)),
            pl.BlockSpec((H, D), lambda i: (0, 0)),
            pl.BlockSpec((1, D), lambda i: (0, 0)),
            pl.BlockSpec((1, D), lambda i: (0, 0)),
        ],
        out_specs=[
            pl.BlockSpec((128, D), lambda i: (i, 0)),
            pl.BlockSpec((128, 2 * H), lambda i: (i, 0)),
        ],
        out_shape=[
            jax.ShapeDtypeStruct((NP, D), F32),
            jax.ShapeDtypeStruct((NP, 2 * H), F32),
        ],
    )(accs, h, asd, W, acat, eh, scale, shift)


def _tc_last(accs, h, asd, W3p, eh, scale, shift):
    return pl.pallas_call(
        _tc_last_body,
        grid=(NB,),
        in_specs=[
            pl.BlockSpec((2, 128, ACCW), lambda i: (0, i, 0)),
            pl.BlockSpec((128, D), lambda i: (i, 0)),
            pl.BlockSpec((128, 2 * H), lambda i: (i, 0)),
            pl.BlockSpec((D, D), lambda i: (0, 0)),
            pl.BlockSpec((H, D), lambda i: (0, 0)),
            pl.BlockSpec((1, D), lambda i: (0, 0)),
            pl.BlockSpec((1, D), lambda i: (0, 0)),
        ],
        out_specs=pl.BlockSpec((128, D), lambda i: (i, 0)),
        out_shape=jax.ShapeDtypeStruct((NP, D), F32),
    )(accs, h, asd, W3p, eh, scale, shift)


def _tc4(num0, num1, den0, den1, h2d, al2d, b3s):
    return pl.pallas_call(
        _tc4_body,
        out_shape=jax.ShapeDtypeStruct((NB, 128), F32),
    )(num0, num1, den0, den1, h2d, al2d, b3s)


def _vperm(x, idx):
    return lax.gather(
        x, idx[:, None],
        lax.GatherDimensionNumbers(offset_dims=(), collapsed_slice_dims=(0,),
                                   start_index_map=(0,)),
        (1,), mode=lax.GatherScatterMode.PROMISE_IN_BOUNDS)


def _sc_edge_body(src_hbm, dst_hbm, asd_hbm, h_hbm, zeros_hbm, out_hbm,
                  idx_s2, idx_d2, asd_s2, asd_d2, hrows2, contrib,
                  acc_sh, semg0, semg1, semi0, semi1):
    c = lax.axis_index("c")
    s = lax.axis_index("s")
    wid = s * 2 + c
    r0 = s * ROWS_PER_TILE
    pltpu.sync_copy(zeros_hbm.at[pl.ds(r0, ROWS_PER_TILE)],
                    acc_sh.at[pl.ds(r0, ROWS_PER_TILE)])
    plsc.subcore_barrier()

    shift_idx = (lax.iota(jnp.int32, 16) + 8) & 15
    splat = [jnp.full((16,), hh, jnp.int32) for hh in range(H)]
    semg = (semg0, semg1)
    semi = (semi0, semi1)

    def issue(gb, sl):
        return (pltpu.async_copy(asd_hbm.at[idx_s2.at[sl]], asd_s2.at[gb],
                                 semg[gb]),
                pltpu.async_copy(asd_hbm.at[idx_d2.at[sl]], asd_d2.at[gb],
                                 semg[gb]),
                pltpu.async_copy(h_hbm.at[idx_s2.at[sl]], hrows2.at[gb],
                                 semg[gb]))

    def drain(cps):
        for cp in cps:
            cp.wait()

    for b in range(2):
        pltpu.sync_copy(src_hbm.at[wid, b], idx_s2.at[b])
        pltpu.sync_copy(dst_hbm.at[wid, b], idx_d2.at[b])
    gather_spec = [issue(0, 0), issue(1, 1)]

    def outer_body(io, carry):
        for b in range(4):
            i = io * 4 + b
            gb = b % 2
            sl = (b + 2) % 4
            drain(gather_spec[gb])
            inext = lax.rem(i + 2, NBATCH)
            cpi = (pltpu.async_copy(src_hbm.at[wid, inext], idx_s2.at[sl],
                                    semi[0]),
                   pltpu.async_copy(dst_hbm.at[wid, inext], idx_d2.at[sl],
                                    semi[0]))

            @plsc.parallel_loop(0, EB, unroll=4)
            def _edges(e):
                rs = asd_s2[gb, e, :]
                rd = asd_d2[gb, e, :]
                rd8 = _vperm(rd, shift_idx)
                alpha = rs + rd8
                w = jnp.exp(jnp.maximum(alpha, 0.2 * alpha))
                contrib[e, pl.ds(D, 16)] = w
                for hh in range(H):
                    wv = _vperm(w, splat[hh])
                    contrib[e, pl.ds(hh * 16, 16)] = (
                        wv * hrows2[gb, e, pl.ds(hh * 16, 16)])
            pltpu.sync_copy(contrib, acc_sh.at[idx_d2.at[b]],
                            add=True)
            drain(cpi)
            issue(gb, sl)
        return carry

    lax.fori_loop(0, NBATCH // 4, outer_body, 0)
    drain(gather_spec[0])
    drain(gather_spec[1])
    plsc.subcore_barrier()
    pltpu.sync_copy(acc_sh.at[pl.ds(r0, ROWS_PER_TILE)],
                    out_hbm.at[c, pl.ds(r0, ROWS_PER_TILE)])


def _sc_edge3_body(src_hbm, dst_hbm, h3_hbm, att_hbm, zeros_hbm, out_hbm,
                   idx_s, idx_d, h3_v, att_v, contrib, acc_sh, sem):
    c = lax.axis_index("c")
    s = lax.axis_index("s")
    wid = s * 2 + c
    r0 = s * ROWS_PER_TILE
    pltpu.sync_copy(zeros_hbm.at[pl.ds(r0, ROWS_PER_TILE)],
                    acc_sh.at[pl.ds(r0, ROWS_PER_TILE)])
    pltpu.sync_copy(h3_hbm, h3_v)
    pltpu.sync_copy(att_hbm, att_v)
    plsc.subcore_barrier()

    sv = att_v[0, :]
    dv = att_v[1, :]
    zcol = jnp.zeros((16,), jnp.int32)
    ocol = jnp.full((16,), 1, jnp.int32)
    zero16 = jnp.zeros((16,), F32)

    def zrow(r, carry):
        contrib[r, :] = zero16
        return carry

    lax.fori_loop(0, EB, zrow, 0)

    def batch_body(i, carry):
        pltpu.sync_copy(src_hbm.at[wid, i], idx_s)
        pltpu.sync_copy(dst_hbm.at[wid, i], idx_d)
        for g in range(EB // 16):
            src16 = idx_s[pl.ds(g * 16, 16)]
            dst16 = idx_d[pl.ds(g * 16, 16)]
            hsrc = plsc.load_gather(h3_v, [src16])
            hdst = plsc.load_gather(h3_v, [dst16])
            al = sv * hsrc + dv * hdst
            w = jnp.exp(jnp.maximum(al, 0.2 * al))
            eid = lax.iota(jnp.int32, 16) + g * 16
            plsc.store_scatter(contrib, [eid, zcol], w * hsrc)
            plsc.store_scatter(contrib, [eid, ocol], w)
        pltpu.sync_copy(contrib, acc_sh.at[idx_d], add=True)
        return carry

    lax.fori_loop(0, NBATCH, batch_body, 0)
    plsc.subcore_barrier()
    pltpu.sync_copy(acc_sh.at[pl.ds(r0, ROWS_PER_TILE)],
                    out_hbm.at[c, pl.ds(r0, ROWS_PER_TILE)])


@functools.lru_cache(maxsize=None)
def _sc_kernels():
    mesh = plsc.VectorSubcoreMesh(core_axis_name="c", subcore_axis_name="s",
                                  num_cores=2, num_subcores=16)
    params = pltpu.CompilerParams(use_tc_tiling_on_sc=False,
                                  needs_layout_passes=False)
    sc_edge = pl.kernel(
        _sc_edge_body,
        out_type=jax.ShapeDtypeStruct((2, NP, ACCW), F32),
        mesh=mesh,
        compiler_params=params,
        scratch_types=[
            pltpu.VMEM((4, EB), jnp.int32),
            pltpu.VMEM((4, EB), jnp.int32),
            pltpu.VMEM((2, EB, 16), F32),
            pltpu.VMEM((2, EB, 16), F32),
            pltpu.VMEM((2, EB, D), F32),
            pltpu.VMEM((EB, ACCW), F32),
            pltpu.VMEM_SHARED((NP, ACCW), F32),
            pltpu.SemaphoreType.DMA,
            pltpu.SemaphoreType.DMA,
            pltpu.SemaphoreType.DMA,
            pltpu.SemaphoreType.DMA,
        ],
    )
    sc_edge3 = pl.kernel(
        _sc_edge3_body,
        out_type=jax.ShapeDtypeStruct((2, NP, 16), F32),
        mesh=mesh,
        compiler_params=params,
        scratch_types=[
            pltpu.VMEM((EB,), jnp.int32),
            pltpu.VMEM((EB,), jnp.int32),
            pltpu.VMEM((NP,), F32),
            pltpu.VMEM((2, 16), F32),
            pltpu.VMEM((EB, 16), F32),
            pltpu.VMEM_SHARED((NP, 16), F32),
            pltpu.SemaphoreType.DMA,
        ],
    )
    return sc_edge, sc_edge3


def _acat(asrc, adst):
    eye = jnp.eye(H, dtype=F32)
    a_s = (asrc[:, :, None] * eye[:, None, :]).reshape(H * HD, H)
    a_d = (adst[:, :, None] * eye[:, None, :]).reshape(H * HD, H)
    return jnp.concatenate([a_s, a_d], axis=1)


def kernel(x, edge_index, W1, a_src1, a_dst1, b1, g1, be1,
           W2, a_src2, a_dst2, b2, g2, be2,
           W3, a_src3, a_dst3, b3):
    xp = jnp.zeros((NP, D), F32).at[:N_NODES].set(x)
    src = jnp.zeros((E_PAD,), jnp.int32).at[:N_EDGES].set(
        edge_index[0]).reshape(NWORK, NBATCH, EB)
    dst = jnp.full((E_PAD,), NP - 1, jnp.int32).at[:N_EDGES].set(
        edge_index[1]).reshape(NWORK, NBATCH, EB)

    acat1 = _acat(a_src1, a_dst1)
    acat2 = _acat(a_src2, a_dst2)
    eh = (jnp.eye(H, dtype=F32)[:, :, None]
          * jnp.ones((HD,), F32)).reshape(H, H * HD)
    inv = 1.0 / jnp.sqrt(jnp.float32(1.0 + 1e-5))
    sc1 = (g1 * inv).reshape(1, D)
    sh1 = (b1 * g1 * inv + be1).reshape(1, D)
    sc2 = (g2 * inv).reshape(1, D)
    sh2 = (b2 * g2 * inv + be2).reshape(1, D)

    s3 = a_src3[0, 0]
    d3 = a_dst3[0, 0]
    W3p = jnp.zeros((D, D), F32)
    W3p = W3p.at[:, 0].set(W3[:, 0])
    W3p = W3p.at[:, 1].set(W3[:, 0] * (s3 + d3))
    att3 = jnp.stack([jnp.full((16,), s3, F32), jnp.full((16,), d3, F32)])

    zeros_acc = jnp.zeros((NP, ACCW), F32)
    zeros_a3 = jnp.zeros((NP, 16), F32)
    _sc_edge, _sc_edge3 = _sc_kernels()

    h1, asd1 = _tc1(xp, W1, acat1)
    accs1 = _sc_edge(src, dst, asd1, h1, zeros_acc)
    h2, asd2 = _tc_mid(accs1, h1, asd1, W2, acat2, eh, sc1, sh1)
    accs2 = _sc_edge(src, dst, asd2, h2, zeros_acc)
    h3full = _tc_last(accs2, h2, asd2, W3p, eh, sc2, sh2)

    h3 = h3full[:, 0]
    al3 = h3full[:, 1]
    accs3 = _sc_edge3(src, dst, h3, att3, zeros_a3)

    num0 = accs3[0, :, 0].reshape(NB, 128)
    num1 = accs3[1, :, 0].reshape(NB, 128)
    den0 = accs3[0, :, 1].reshape(NB, 128)
    den1 = accs3[1, :, 1].reshape(NB, 128)
    h2d = h3.reshape(NB, 128)
    al2d = al3.reshape(NB, 128)
    b3s = jnp.broadcast_to(b3.reshape(1, 1), (1, 128)).astype(F32)

    out2d = _tc4(num0, num1, den0, den1, h2d, al2d, b3s)
    return out2d.reshape(NP, 1)[:N_NODES]


# speedup vs baseline: 1.1492x; 1.0003x over previous
import functools

import jax
import jax.numpy as jnp
from jax import lax
from jax.experimental import pallas as pl
from jax.experimental.pallas import tpu as pltpu
from jax.experimental.pallas import tpu_sc as plsc

N_NODES = 10000
N_EDGES = 320000
D = 128
H = 8
HD = 16
NP = 10112
NB = NP // 128
ACCW = 144
NWORK = 32
EB = 80
NBATCH = 128
EPW = NBATCH * EB
E_PAD = NWORK * EPW
ROWS_PER_TILE = NP // 16
F32 = jnp.float32
_PREC = lax.Precision.HIGHEST


def _tc1_body(x_ref, w_ref, acat_ref, h_ref, asd_ref):
    h = jnp.dot(x_ref[...], w_ref[...], precision=_PREC)
    h_ref[...] = h
    asd_ref[...] = jnp.dot(h, acat_ref[...], precision=_PREC)


def _normalize(accs, h, asd, eh, scale, shift):
    a = accs[0] + accs[1]
    num = a[:, :D]
    den = a[:, D:D + H]
    al = asd[:, :H] + asd[:, H:2 * H]
    w_self = jnp.exp(jnp.maximum(al, 0.2 * al))
    wexp = jnp.dot(w_self, eh, precision=_PREC)
    dexp = jnp.dot(den + w_self, eh, precision=_PREC)
    y = (num + wexp * h) / dexp
    y = y * scale + shift
    return jnp.where(y > 0, y, jnp.exp(y) - 1.0)


def _tc_mid_body(accs_ref, h_ref, asd_ref, w_ref, acat_ref, eh_ref, sc_ref,
                 sh_ref, h2_ref, asd2_ref):
    y = _normalize(accs_ref[...], h_ref[...], asd_ref[...], eh_ref[...],
                   sc_ref[...], sh_ref[...])
    h2 = jnp.dot(y, w_ref[...], precision=_PREC)
    h2_ref[...] = h2
    asd2_ref[...] = jnp.dot(h2, acat_ref[...], precision=_PREC)


def _tc_last_body(accs_ref, h_ref, asd_ref, w_ref, eh_ref, sc_ref, sh_ref,
                  h3_ref):
    y = _normalize(accs_ref[...], h_ref[...], asd_ref[...], eh_ref[...],
                   sc_ref[...], sh_ref[...])
    h3_ref[...] = jnp.dot(y, w_ref[...], precision=_PREC)


def _tc4_body(num0_ref, num1_ref, den0_ref, den1_ref, h_ref, al_ref, b3_ref,
              out_ref):
    num = num0_ref[...] + num1_ref[...]
    den = den0_ref[...] + den1_ref[...]
    al = al_ref[...]
    w_self = jnp.exp(jnp.maximum(al, 0.2 * al))
    r = (num + w_self * h_ref[...]) / (den + w_self) + b3_ref[...]
    out_ref[...] = 1.0 / (1.0 + jnp.exp(-r))


def _tc1(xp, W1, acat1):
    return pl.pallas_call(
        _tc1_body,
        grid=(NB,),
        in_specs=[
            pl.BlockSpec((128, D), lambda i: (i, 0)),
            pl.BlockSpec((D, D), lambda i: (0, 0)),
            pl.BlockSpec((D, 2 * H), lambda i: (0, 0)),
        ],
        out_specs=[
            pl.BlockSpec((128, D), lambda i: (i, 0)),
            pl.BlockSpec((128, 2 * H), lambda i: (i, 0)),
        ],
        out_shape=[
            jax.ShapeDtypeStruct((NP, D), F32),
            jax.ShapeDtypeStruct((NP, 2 * H), F32),
        ],
    )(xp, W1, acat1)


def _tc_mid(accs, h, asd, W, acat, eh, scale, shift):
    return pl.pallas_call(
        _tc_mid_body,
        grid=(NB,),
        in_specs=[
            pl.BlockSpec((2, 128, ACCW), lambda i: (0, i, 0)),
            pl.BlockSpec((128, D), lambda i: (i, 0)),
            pl.BlockSpec((128, 2 * H), lambda i: (i, 0)),
            pl.BlockSpec((D, D), lambda i: (0, 0)),
            pl.BlockSpec((D, 2 * H), lambda i: (0, 0)),
            pl.BlockSpec((H, D), lambda i: (0, 0)),
            pl.BlockSpec((1, D), lambda i: (0, 0)),
            pl.BlockSpec((1, D), lambda i: (0, 0)),
        ],
        out_specs=[
            pl.BlockSpec((128, D), lambda i: (i, 0)),
            pl.BlockSpec((128, 2 * H), lambda i: (i, 0)),
        ],
        out_shape=[
            jax.ShapeDtypeStruct((NP, D), F32),
            jax.ShapeDtypeStruct((NP, 2 * H), F32),
        ],
    )(accs, h, asd, W, acat, eh, scale, shift)


def _tc_last(accs, h, asd, W3p, eh, scale, shift):
    return pl.pallas_call(
        _tc_last_body,
        grid=(NB,),
        in_specs=[
            pl.BlockSpec((2, 128, ACCW), lambda i: (0, i, 0)),
            pl.BlockSpec((128, D), lambda i: (i, 0)),
            pl.BlockSpec((128, 2 * H), lambda i: (i, 0)),
            pl.BlockSpec((D, D), lambda i: (0, 0)),
            pl.BlockSpec((H, D), lambda i: (0, 0)),
            pl.BlockSpec((1, D), lambda i: (0, 0)),
            pl.BlockSpec((1, D), lambda i: (0, 0)),
        ],
        out_specs=pl.BlockSpec((128, D), lambda i: (i, 0)),
        out_shape=jax.ShapeDtypeStruct((NP, D), F32),
    )(accs, h, asd, W3p, eh, scale, shift)


def _tc4(num0, num1, den0, den1, h2d, al2d, b3s):
    return pl.pallas_call(
        _tc4_body,
        out_shape=jax.ShapeDtypeStruct((NB, 128), F32),
    )(num0, num1, den0, den1, h2d, al2d, b3s)


def _vperm(x, idx):
    return lax.gather(
        x, idx[:, None],
        lax.GatherDimensionNumbers(offset_dims=(), collapsed_slice_dims=(0,),
                                   start_index_map=(0,)),
        (1,), mode=lax.GatherScatterMode.PROMISE_IN_BOUNDS)


def _sc_edge_body(src_hbm, dst_hbm, asd_hbm, h_hbm, zeros_hbm, out_hbm,
                  idx_s2, idx_d2, asd_s2, asd_d2, hrows2, contrib,
                  acc_sh, semg0, semg1, semi0, semi1):
    c = lax.axis_index("c")
    s = lax.axis_index("s")
    wid = s * 2 + c
    r0 = s * ROWS_PER_TILE
    pltpu.sync_copy(zeros_hbm.at[pl.ds(r0, ROWS_PER_TILE)],
                    acc_sh.at[pl.ds(r0, ROWS_PER_TILE)])
    plsc.subcore_barrier()

    shift_idx = (lax.iota(jnp.int32, 16) + 8) & 15
    splat = [jnp.full((16,), hh, jnp.int32) for hh in range(H)]
    semg = (semg0, semg1)
    semi = (semi0, semi1)

    def issue(gb, sl):
        return (pltpu.async_copy(asd_hbm.at[idx_s2.at[sl]], asd_s2.at[gb],
                                 semg[gb]),
                pltpu.async_copy(asd_hbm.at[idx_d2.at[sl]], asd_d2.at[gb],
                                 semg[gb]),
                pltpu.async_copy(h_hbm.at[idx_s2.at[sl]], hrows2.at[gb],
                                 semg[gb]))

    def drain(cps):
        for cp in cps:
            cp.wait()

    for b in range(2):
        pltpu.sync_copy(src_hbm.at[wid, b], idx_s2.at[b])
        pltpu.sync_copy(dst_hbm.at[wid, b], idx_d2.at[b])
    gather_spec = [issue(0, 0), issue(1, 1)]

    def outer_body(io, carry):
        for b in range(4):
            i = io * 4 + b
            gb = b % 2
            sl = (b + 2) % 4
            drain(gather_spec[gb])
            inext = lax.rem(i + 2, NBATCH)
            cpi = (pltpu.async_copy(src_hbm.at[wid, inext], idx_s2.at[sl],
                                    semi[0]),
                   pltpu.async_copy(dst_hbm.at[wid, inext], idx_d2.at[sl],
                                    semi[0]))

            @plsc.parallel_loop(0, EB, unroll=8)
            def _edges(e):
                rs = asd_s2[gb, e, :]
                rd = asd_d2[gb, e, :]
                rd8 = _vperm(rd, shift_idx)
                alpha = rs + rd8
                w = jnp.exp(jnp.maximum(alpha, 0.2 * alpha))
                contrib[e, pl.ds(D, 16)] = w
                for hh in range(H):
                    wv = _vperm(w, splat[hh])
                    contrib[e, pl.ds(hh * 16, 16)] = (
                        wv * hrows2[gb, e, pl.ds(hh * 16, 16)])
            pltpu.sync_copy(contrib, acc_sh.at[idx_d2.at[b]],
                            add=True)
            drain(cpi)
            issue(gb, sl)
        return carry

    lax.fori_loop(0, NBATCH // 4, outer_body, 0)
    drain(gather_spec[0])
    drain(gather_spec[1])
    plsc.subcore_barrier()
    pltpu.sync_copy(acc_sh.at[pl.ds(r0, ROWS_PER_TILE)],
                    out_hbm.at[c, pl.ds(r0, ROWS_PER_TILE)])


def _sc_edge3_body(src_hbm, dst_hbm, h3_hbm, att_hbm, zeros_hbm, out_hbm,
                   idx_s, idx_d, h3_v, att_v, contrib, acc_sh, sem):
    c = lax.axis_index("c")
    s = lax.axis_index("s")
    wid = s * 2 + c
    r0 = s * ROWS_PER_TILE
    pltpu.sync_copy(zeros_hbm.at[pl.ds(r0, ROWS_PER_TILE)],
                    acc_sh.at[pl.ds(r0, ROWS_PER_TILE)])
    pltpu.sync_copy(h3_hbm, h3_v)
    pltpu.sync_copy(att_hbm, att_v)
    plsc.subcore_barrier()

    sv = att_v[0, :]
    dv = att_v[1, :]
    zcol = jnp.zeros((16,), jnp.int32)
    ocol = jnp.full((16,), 1, jnp.int32)
    zero16 = jnp.zeros((16,), F32)

    def zrow(r, carry):
        contrib[r, :] = zero16
        return carry

    lax.fori_loop(0, EB, zrow, 0)

    def batch_body(i, carry):
        pltpu.sync_copy(src_hbm.at[wid, i], idx_s)
        pltpu.sync_copy(dst_hbm.at[wid, i], idx_d)
        for g in range(EB // 16):
            src16 = idx_s[pl.ds(g * 16, 16)]
            dst16 = idx_d[pl.ds(g * 16, 16)]
            hsrc = plsc.load_gather(h3_v, [src16])
            hdst = plsc.load_gather(h3_v, [dst16])
            al = sv * hsrc + dv * hdst
            w = jnp.exp(jnp.maximum(al, 0.2 * al))
            eid = lax.iota(jnp.int32, 16) + g * 16
            plsc.store_scatter(contrib, [eid, zcol], w * hsrc)
            plsc.store_scatter(contrib, [eid, ocol], w)
        pltpu.sync_copy(contrib, acc_sh.at[idx_d], add=True)
        return carry

    lax.fori_loop(0, NBATCH, batch_body, 0)
    plsc.subcore_barrier()
    pltpu.sync_copy(acc_sh.at[pl.ds(r0, ROWS_PER_TILE)],
                    out_hbm.at[c, pl.ds(r0, ROWS_PER_TILE)])


@functools.lru_cache(maxsize=None)
def _sc_kernels():
    mesh = plsc.VectorSubcoreMesh(core_axis_name="c", subcore_axis_name="s",
                                  num_cores=2, num_subcores=16)
    params = pltpu.CompilerParams(use_tc_tiling_on_sc=False,
                                  needs_layout_passes=False)
    sc_edge = pl.kernel(
        _sc_edge_body,
        out_type=jax.ShapeDtypeStruct((2, NP, ACCW), F32),
        mesh=mesh,
        compiler_params=params,
        scratch_types=[
            pltpu.VMEM((4, EB), jnp.int32),
            pltpu.VMEM((4, EB), jnp.int32),
            pltpu.VMEM((2, EB, 16), F32),
            pltpu.VMEM((2, EB, 16), F32),
            pltpu.VMEM((2, EB, D), F32),
            pltpu.VMEM((EB, ACCW), F32),
            pltpu.VMEM_SHARED((NP, ACCW), F32),
            pltpu.SemaphoreType.DMA,
            pltpu.SemaphoreType.DMA,
            pltpu.SemaphoreType.DMA,
            pltpu.SemaphoreType.DMA,
        ],
    )
    sc_edge3 = pl.kernel(
        _sc_edge3_body,
        out_type=jax.ShapeDtypeStruct((2, NP, 16), F32),
        mesh=mesh,
        compiler_params=params,
        scratch_types=[
            pltpu.VMEM((EB,), jnp.int32),
            pltpu.VMEM((EB,), jnp.int32),
            pltpu.VMEM((NP,), F32),
            pltpu.VMEM((2, 16), F32),
            pltpu.VMEM((EB, 16), F32),
            pltpu.VMEM_SHARED((NP, 16), F32),
            pltpu.SemaphoreType.DMA,
        ],
    )
    return sc_edge, sc_edge3


def _acat(asrc, adst):
    eye = jnp.eye(H, dtype=F32)
    a_s = (asrc[:, :, None] * eye[:, None, :]).reshape(H * HD, H)
    a_d = (adst[:, :, None] * eye[:, None, :]).reshape(H * HD, H)
    return jnp.concatenate([a_s, a_d], axis=1)


def kernel(x, edge_index, W1, a_src1, a_dst1, b1, g1, be1,
           W2, a_src2, a_dst2, b2, g2, be2,
           W3, a_src3, a_dst3, b3):
    xp = jnp.zeros((NP, D), F32).at[:N_NODES].set(x)
    src = jnp.zeros((E_PAD,), jnp.int32).at[:N_EDGES].set(
        edge_index[0]).reshape(NWORK, NBATCH, EB)
    dst = jnp.full((E_PAD,), NP - 1, jnp.int32).at[:N_EDGES].set(
        edge_index[1]).reshape(NWORK, NBATCH, EB)

    acat1 = _acat(a_src1, a_dst1)
    acat2 = _acat(a_src2, a_dst2)
    eh = (jnp.eye(H, dtype=F32)[:, :, None]
          * jnp.ones((HD,), F32)).reshape(H, H * HD)
    inv = 1.0 / jnp.sqrt(jnp.float32(1.0 + 1e-5))
    sc1 = (g1 * inv).reshape(1, D)
    sh1 = (b1 * g1 * inv + be1).reshape(1, D)
    sc2 = (g2 * inv).reshape(1, D)
    sh2 = (b2 * g2 * inv + be2).reshape(1, D)

    s3 = a_src3[0, 0]
    d3 = a_dst3[0, 0]
    W3p = jnp.zeros((D, D), F32)
    W3p = W3p.at[:, 0].set(W3[:, 0])
    W3p = W3p.at[:, 1].set(W3[:, 0] * (s3 + d3))
    att3 = jnp.stack([jnp.full((16,), s3, F32), jnp.full((16,), d3, F32)])

    zeros_acc = jnp.zeros((NP, ACCW), F32)
    zeros_a3 = jnp.zeros((NP, 16), F32)
    _sc_edge, _sc_edge3 = _sc_kernels()

    h1, asd1 = _tc1(xp, W1, acat1)
    accs1 = _sc_edge(src, dst, asd1, h1, zeros_acc)
    h2, asd2 = _tc_mid(accs1, h1, asd1, W2, acat2, eh, sc1, sh1)
    accs2 = _sc_edge(src, dst, asd2, h2, zeros_acc)
    h3full = _tc_last(accs2, h2, asd2, W3p, eh, sc2, sh2)

    h3 = h3full[:, 0]
    al3 = h3full[:, 1]
    accs3 = _sc_edge3(src, dst, h3, att3, zeros_a3)

    num0 = accs3[0, :, 0].reshape(NB, 128)
    num1 = accs3[1, :, 0].reshape(NB, 128)
    den0 = accs3[0, :, 1].reshape(NB, 128)
    den1 = accs3[1, :, 1].reshape(NB, 128)
    h2d = h3.reshape(NB, 128)
    al2d = al3.reshape(NB, 128)
    b3s = jnp.broadcast_to(b3.reshape(1, 1), (1, 128)).astype(F32)

    out2d = _tc4(num0, num1, den0, den1, h2d, al2d, b3s)
    return out2d.reshape(NP, 1)[:N_NODES]


# speedup vs baseline: 1.1493x; 1.0001x over previous
import functools

import jax
import jax.numpy as jnp
from jax import lax
from jax.experimental import pallas as pl
from jax.experimental.pallas import tpu as pltpu
from jax.experimental.pallas import tpu_sc as plsc

N_NODES = 10000
N_EDGES = 320000
D = 128
H = 8
HD = 16
NP = 10112
NB = NP // 128
ACCW = 144
NWORK = 32
EB = 80
NBATCH = 128
EPW = NBATCH * EB
E_PAD = NWORK * EPW
ROWS_PER_TILE = NP // 16
F32 = jnp.float32
_PREC = lax.Precision.HIGHEST


def _tc1_body(x_ref, w_ref, acat_ref, h_ref, asd_ref):
    h = jnp.dot(x_ref[...], w_ref[...], precision=_PREC)
    h_ref[...] = h
    asd_ref[...] = jnp.dot(h, acat_ref[...], precision=_PREC)


def _normalize(accs, h, asd, eh, scale, shift):
    a = accs[0] + accs[1]
    num = a[:, :D]
    den = a[:, D:D + H]
    al = asd[:, :H] + asd[:, H:2 * H]
    w_self = jnp.exp(jnp.maximum(al, 0.2 * al))
    wexp = jnp.dot(w_self, eh, precision=_PREC)
    dexp = jnp.dot(den + w_self, eh, precision=_PREC)
    y = (num + wexp * h) / dexp
    y = y * scale + shift
    return jnp.where(y > 0, y, jnp.exp(y) - 1.0)


def _tc_mid_body(accs_ref, h_ref, asd_ref, w_ref, acat_ref, eh_ref, sc_ref,
                 sh_ref, h2_ref, asd2_ref):
    y = _normalize(accs_ref[...], h_ref[...], asd_ref[...], eh_ref[...],
                   sc_ref[...], sh_ref[...])
    h2 = jnp.dot(y, w_ref[...], precision=_PREC)
    h2_ref[...] = h2
    asd2_ref[...] = jnp.dot(h2, acat_ref[...], precision=_PREC)


def _tc_last_body(accs_ref, h_ref, asd_ref, w_ref, eh_ref, sc_ref, sh_ref,
                  h3_ref):
    y = _normalize(accs_ref[...], h_ref[...], asd_ref[...], eh_ref[...],
                   sc_ref[...], sh_ref[...])
    h3_ref[...] = jnp.dot(y, w_ref[...], precision=_PREC)


def _tc4_body(num0_ref, num1_ref, den0_ref, den1_ref, h_ref, al_ref, b3_ref,
              out_ref):
    num = num0_ref[...] + num1_ref[...]
    den = den0_ref[...] + den1_ref[...]
    al = al_ref[...]
    w_self = jnp.exp(jnp.maximum(al, 0.2 * al))
    r = (num + w_self * h_ref[...]) / (den + w_self) + b3_ref[...]
    out_ref[...] = 1.0 / (1.0 + jnp.exp(-r))


def _tc1(xp, W1, acat1):
    return pl.pallas_call(
        _tc1_body,
        grid=(NB,),
        in_specs=[
            pl.BlockSpec((128, D), lambda i: (i, 0)),
            pl.BlockSpec((D, D), lambda i: (0, 0)),
            pl.BlockSpec((D, 2 * H), lambda i: (0, 0)),
        ],
        out_specs=[
            pl.BlockSpec((128, D), lambda i: (i, 0)),
            pl.BlockSpec((128, 2 * H), lambda i: (i, 0)),
        ],
        out_shape=[
            jax.ShapeDtypeStruct((NP, D), F32),
            jax.ShapeDtypeStruct((NP, 2 * H), F32),
        ],
    )(xp, W1, acat1)


def _tc_mid(accs, h, asd, W, acat, eh, scale, shift):
    return pl.pallas_call(
        _tc_mid_body,
        grid=(NB,),
        in_specs=[
            pl.BlockSpec((2, 128, ACCW), lambda i: (0, i, 0)),
            pl.BlockSpec((128, D), lambda i: (i, 0)),
            pl.BlockSpec((128, 2 * H), lambda i: (i, 0)),
            pl.BlockSpec((D, D), lambda i: (0, 0)),
            pl.BlockSpec((D, 2 * H), lambda i: (0, 0)),
            pl.BlockSpec((H, D), lambda i: (0, 0)),
            pl.BlockSpec((1, D), lambda i: (0, 0)),
            pl.BlockSpec((1, D), lambda i: (0, 0)),
        ],
        out_specs=[
            pl.BlockSpec((128, D), lambda i: (i, 0)),
            pl.BlockSpec((128, 2 * H), lambda i: (i, 0)),
        ],
        out_shape=[
            jax.ShapeDtypeStruct((NP, D), F32),
            jax.ShapeDtypeStruct((NP, 2 * H), F32),
        ],
    )(accs, h, asd, W, acat, eh, scale, shift)


def _tc_last(accs, h, asd, W3p, eh, scale, shift):
    return pl.pallas_call(
        _tc_last_body,
        grid=(NB,),
        in_specs=[
            pl.BlockSpec((2, 128, ACCW), lambda i: (0, i, 0)),
            pl.BlockSpec((128, D), lambda i: (i, 0)),
            pl.BlockSpec((128, 2 * H), lambda i: (i, 0)),
            pl.BlockSpec((D, D), lambda i: (0, 0)),
            pl.BlockSpec((H, D), lambda i: (0, 0)),
            pl.BlockSpec((1, D), lambda i: (0, 0)),
            pl.BlockSpec((1, D), lambda i: (0, 0)),
        ],
        out_specs=pl.BlockSpec((128, D), lambda i: (i, 0)),
        out_shape=jax.ShapeDtypeStruct((NP, D), F32),
    )(accs, h, asd, W3p, eh, scale, shift)


def _tc4(num0, num1, den0, den1, h2d, al2d, b3s):
    return pl.pallas_call(
        _tc4_body,
        out_shape=jax.ShapeDtypeStruct((NB, 128), F32),
    )(num0, num1, den0, den1, h2d, al2d, b3s)


def _vperm(x, idx):
    return lax.gather(
        x, idx[:, None],
        lax.GatherDimensionNumbers(offset_dims=(), collapsed_slice_dims=(0,),
                                   start_index_map=(0,)),
        (1,), mode=lax.GatherScatterMode.PROMISE_IN_BOUNDS)


def _sc_edge_body(src_hbm, dst_hbm, asd_hbm, h_hbm, zeros_hbm, out_hbm,
                  idx_s2, idx_d2, asd_s2, asd_d2, hrows2, contrib,
                  acc_sh, semg0, semg1, semi0, semi1):
    c = lax.axis_index("c")
    s = lax.axis_index("s")
    wid = s * 2 + c
    r0 = s * ROWS_PER_TILE
    pltpu.sync_copy(zeros_hbm.at[pl.ds(r0, ROWS_PER_TILE)],
                    acc_sh.at[pl.ds(r0, ROWS_PER_TILE)])
    plsc.subcore_barrier()

    shift_idx = (lax.iota(jnp.int32, 16) + 8) & 15
    splat = [jnp.full((16,), hh, jnp.int32) for hh in range(H)]
    semg = (semg0, semg1)
    semi = (semi0, semi1)

    def issue(gb, sl):
        return (pltpu.async_copy(asd_hbm.at[idx_s2.at[sl]], asd_s2.at[gb],
                                 semg[gb]),
                pltpu.async_copy(asd_hbm.at[idx_d2.at[sl]], asd_d2.at[gb],
                                 semg[gb]),
                pltpu.async_copy(h_hbm.at[idx_s2.at[sl]], hrows2.at[gb],
                                 semg[gb]))

    def drain(cps):
        for cp in cps:
            cp.wait()

    for b in range(2):
        pltpu.sync_copy(src_hbm.at[wid, b], idx_s2.at[b])
        pltpu.sync_copy(dst_hbm.at[wid, b], idx_d2.at[b])
    gather_spec = [issue(0, 0), issue(1, 1)]

    def outer_body(io, carry):
        for b in range(4):
            i = io * 4 + b
            gb = b % 2
            sl = (b + 2) % 4
            drain(gather_spec[gb])
            inext = lax.rem(i + 2, NBATCH)
            cpi = (pltpu.async_copy(src_hbm.at[wid, inext], idx_s2.at[sl],
                                    semi[0]),
                   pltpu.async_copy(dst_hbm.at[wid, inext], idx_d2.at[sl],
                                    semi[0]))

            @plsc.parallel_loop(0, EB, unroll=4)
            def _edges(e):
                rs = asd_s2[gb, e, :]
                rd = asd_d2[gb, e, :]
                rd8 = _vperm(rd, shift_idx)
                alpha = rs + rd8
                w = jnp.exp(jnp.maximum(alpha, 0.2 * alpha))
                contrib[e, pl.ds(D, 16)] = w
                for hh in range(H):
                    wv = _vperm(w, splat[hh])
                    contrib[e, pl.ds(hh * 16, 16)] = (
                        wv * hrows2[gb, e, pl.ds(hh * 16, 16)])
            pltpu.sync_copy(contrib, acc_sh.at[idx_d2.at[b]],
                            add=True)
            drain(cpi)
            issue(gb, sl)
        return carry

    lax.fori_loop(0, NBATCH // 4, outer_body, 0)
    drain(gather_spec[0])
    drain(gather_spec[1])
    plsc.subcore_barrier()
    pltpu.sync_copy(acc_sh.at[pl.ds(r0, ROWS_PER_TILE)],
                    out_hbm.at[c, pl.ds(r0, ROWS_PER_TILE)])


def _sc_edge3_body(src_hbm, dst_hbm, h3_hbm, att_hbm, zeros_hbm, out_hbm,
                   idx_s, idx_d, h3_v, att_v, contrib, acc_sh, sem):
    c = lax.axis_index("c")
    s = lax.axis_index("s")
    wid = s * 2 + c
    r0 = s * ROWS_PER_TILE
    pltpu.sync_copy(zeros_hbm.at[pl.ds(r0, ROWS_PER_TILE)],
                    acc_sh.at[pl.ds(r0, ROWS_PER_TILE)])
    pltpu.sync_copy(h3_hbm, h3_v)
    pltpu.sync_copy(att_hbm, att_v)
    plsc.subcore_barrier()

    sv = att_v[0, :]
    dv = att_v[1, :]
    zcol = jnp.zeros((16,), jnp.int32)
    ocol = jnp.full((16,), 1, jnp.int32)
    zero16 = jnp.zeros((16,), F32)

    def zrow(r, carry):
        contrib[r, :] = zero16
        return carry

    lax.fori_loop(0, EB, zrow, 0)

    def batch_body(i, carry):
        pltpu.sync_copy(src_hbm.at[wid, i], idx_s)
        pltpu.sync_copy(dst_hbm.at[wid, i], idx_d)
        for g in range(EB // 16):
            src16 = idx_s[pl.ds(g * 16, 16)]
            dst16 = idx_d[pl.ds(g * 16, 16)]
            hsrc = plsc.load_gather(h3_v, [src16])
            hdst = plsc.load_gather(h3_v, [dst16])
            al = sv * hsrc + dv * hdst
            w = jnp.exp(jnp.maximum(al, 0.2 * al))
            eid = lax.iota(jnp.int32, 16) + g * 16
            plsc.store_scatter(contrib, [eid, zcol], w * hsrc)
            plsc.store_scatter(contrib, [eid, ocol], w)
        pltpu.sync_copy(contrib, acc_sh.at[idx_d], add=True)
        return carry

    lax.fori_loop(0, NBATCH, batch_body, 0)
    plsc.subcore_barrier()
    pltpu.sync_copy(acc_sh.at[pl.ds(r0, ROWS_PER_TILE)],
                    out_hbm.at[c, pl.ds(r0, ROWS_PER_TILE)])


@functools.lru_cache(maxsize=None)
def _sc_kernels():
    mesh = plsc.VectorSubcoreMesh(core_axis_name="c", subcore_axis_name="s",
                                  num_cores=2, num_subcores=16)
    params = pltpu.CompilerParams(use_tc_tiling_on_sc=False,
                                  needs_layout_passes=False)
    sc_edge = pl.kernel(
        _sc_edge_body,
        out_type=jax.ShapeDtypeStruct((2, NP, ACCW), F32),
        mesh=mesh,
        compiler_params=params,
        scratch_types=[
            pltpu.VMEM((4, EB), jnp.int32),
            pltpu.VMEM((4, EB), jnp.int32),
            pltpu.VMEM((2, EB, 16), F32),
            pltpu.VMEM((2, EB, 16), F32),
            pltpu.VMEM((2, EB, D), F32),
            pltpu.VMEM((EB, ACCW), F32),
            pltpu.VMEM_SHARED((NP, ACCW), F32),
            pltpu.SemaphoreType.DMA,
            pltpu.SemaphoreType.DMA,
            pltpu.SemaphoreType.DMA,
            pltpu.SemaphoreType.DMA,
        ],
    )
    sc_edge3 = pl.kernel(
        _sc_edge3_body,
        out_type=jax.ShapeDtypeStruct((2, NP, 16), F32),
        mesh=mesh,
        compiler_params=params,
        scratch_types=[
            pltpu.VMEM((EB,), jnp.int32),
            pltpu.VMEM((EB,), jnp.int32),
            pltpu.VMEM((NP,), F32),
            pltpu.VMEM((2, 16), F32),
            pltpu.VMEM((EB, 16), F32),
            pltpu.VMEM_SHARED((NP, 16), F32),
            pltpu.SemaphoreType.DMA,
        ],
    )
    return sc_edge, sc_edge3


def _acat(asrc, adst):
    eye = jnp.eye(H, dtype=F32)
    a_s = (asrc[:, :, None] * eye[:, None, :]).reshape(H * HD, H)
    a_d = (adst[:, :, None] * eye[:, None, :]).reshape(H * HD, H)
    return jnp.concatenate([a_s, a_d], axis=1)


def kernel(x, edge_index, W1, a_src1, a_dst1, b1, g1, be1,
           W2, a_src2, a_dst2, b2, g2, be2,
           W3, a_src3, a_dst3, b3):
    xp = jnp.zeros((NP, D), F32).at[:N_NODES].set(x)
    src = jnp.zeros((E_PAD,), jnp.int32).at[:N_EDGES].set(
        edge_index[0]).reshape(NWORK, NBATCH, EB)
    dst = jnp.full((E_PAD,), NP - 1, jnp.int32).at[:N_EDGES].set(
        edge_index[1]).reshape(NWORK, NBATCH, EB)

    acat1 = _acat(a_src1, a_dst1)
    acat2 = _acat(a_src2, a_dst2)
    eh = (jnp.eye(H, dtype=F32)[:, :, None]
          * jnp.ones((HD,), F32)).reshape(H, H * HD)
    inv = 1.0 / jnp.sqrt(jnp.float32(1.0 + 1e-5))
    sc1 = (g1 * inv).reshape(1, D)
    sh1 = (b1 * g1 * inv + be1).reshape(1, D)
    sc2 = (g2 * inv).reshape(1, D)
    sh2 = (b2 * g2 * inv + be2).reshape(1, D)

    s3 = a_src3[0, 0]
    d3 = a_dst3[0, 0]
    W3p = jnp.zeros((D, D), F32)
    W3p = W3p.at[:, 0].set(W3[:, 0])
    W3p = W3p.at[:, 1].set(W3[:, 0] * (s3 + d3))
    att3 = jnp.stack([jnp.full((16,), s3, F32), jnp.full((16,), d3, F32)])

    zeros_acc = jnp.zeros((NP, ACCW), F32)
    zeros_a3 = jnp.zeros((NP, 16), F32)
    _sc_edge, _sc_edge3 = _sc_kernels()

    h1, asd1 = _tc1(xp, W1, acat1)
    accs1 = _sc_edge(src, dst, asd1, h1, zeros_acc)
    h2, asd2 = _tc_mid(accs1, h1, asd1, W2, acat2, eh, sc1, sh1)
    accs2 = _sc_edge(src, dst, asd2, h2, zeros_acc)
    h3full = _tc_last(accs2, h2, asd2, W3p, eh, sc2, sh2)

    h3 = h3full[:, 0]
    al3 = h3full[:, 1]
    accs3 = _sc_edge3(src, dst, h3, att3, zeros_a3)

    num0 = accs3[0, :, 0].reshape(NB, 128)
    num1 = accs3[1, :, 0].reshape(NB, 128)
    den0 = accs3[0, :, 1].reshape(NB, 128)
    den1 = accs3[1, :, 1].reshape(NB, 128)
    h2d = h3.reshape(NB, 128)
    al2d = al3.reshape(NB, 128)
    b3s = jnp.broadcast_to(b3.reshape(1, 1), (1, 128)).astype(F32)

    out2d = _tc4(num0, num1, den0, den1, h2d, al2d, b3s)
    return out2d.reshape(NP, 1)[:N_NODES]


# speedup vs baseline: 1.8374x; 1.5987x over previous
import functools

import jax
import jax.numpy as jnp
from jax import lax
from jax.experimental import pallas as pl
from jax.experimental.pallas import tpu as pltpu
from jax.experimental.pallas import tpu_sc as plsc

N_NODES = 10000
N_EDGES = 320000
D = 128
H = 8
HD = 16
NP = 10112
NB = NP // 128
ACCW = 144
NWORK = 32
EB = 80
NB_FAST = 192
NB_SLOW = 60
E_PAD = 16 * (NB_FAST + NB_SLOW) * EB
NBATCH3 = E_PAD // (NWORK * EB)
EPW3 = NBATCH3 * EB
ROWS_PER_TILE = NP // 16
F32 = jnp.float32
_PREC = lax.Precision.HIGHEST


def _tc1_body(x_ref, w_ref, acat_ref, h_ref, asd_ref):
    h = jnp.dot(x_ref[...], w_ref[...], precision=_PREC)
    h_ref[...] = h
    asd_ref[...] = jnp.dot(h, acat_ref[...], precision=_PREC)


def _normalize(accs, h, asd, eh, scale, shift):
    a = accs[0] + accs[1]
    num = a[:, :D]
    den = a[:, D:D + H]
    al = asd[:, :H] + asd[:, H:2 * H]
    w_self = jnp.exp(jnp.maximum(al, 0.2 * al))
    wexp = jnp.dot(w_self, eh, precision=_PREC)
    dexp = jnp.dot(den + w_self, eh, precision=_PREC)
    y = (num + wexp * h) / dexp
    y = y * scale + shift
    return jnp.where(y > 0, y, jnp.exp(y) - 1.0)


def _tc_mid_body(accs_ref, h_ref, asd_ref, w_ref, acat_ref, eh_ref, sc_ref,
                 sh_ref, h2_ref, asd2_ref):
    y = _normalize(accs_ref[...], h_ref[...], asd_ref[...], eh_ref[...],
                   sc_ref[...], sh_ref[...])
    h2 = jnp.dot(y, w_ref[...], precision=_PREC)
    h2_ref[...] = h2
    asd2_ref[...] = jnp.dot(h2, acat_ref[...], precision=_PREC)


def _tc_last_body(accs_ref, h_ref, asd_ref, w_ref, eh_ref, sc_ref, sh_ref,
                  h3_ref):
    y = _normalize(accs_ref[...], h_ref[...], asd_ref[...], eh_ref[...],
                   sc_ref[...], sh_ref[...])
    h3_ref[...] = jnp.dot(y, w_ref[...], precision=_PREC)


def _tc4_body(num0_ref, num1_ref, den0_ref, den1_ref, h_ref, al_ref, b3_ref,
              out_ref):
    num = num0_ref[...] + num1_ref[...]
    den = den0_ref[...] + den1_ref[...]
    al = al_ref[...]
    w_self = jnp.exp(jnp.maximum(al, 0.2 * al))
    r = (num + w_self * h_ref[...]) / (den + w_self) + b3_ref[...]
    out_ref[...] = 1.0 / (1.0 + jnp.exp(-r))


def _tc1(xp, W1, acat1):
    return pl.pallas_call(
        _tc1_body,
        grid=(NB,),
        in_specs=[
            pl.BlockSpec((128, D), lambda i: (i, 0)),
            pl.BlockSpec((D, D), lambda i: (0, 0)),
            pl.BlockSpec((D, 2 * H), lambda i: (0, 0)),
        ],
        out_specs=[
            pl.BlockSpec((128, D), lambda i: (i, 0)),
            pl.BlockSpec((128, 2 * H), lambda i: (i, 0)),
        ],
        out_shape=[
            jax.ShapeDtypeStruct((NP, D), F32),
            jax.ShapeDtypeStruct((NP, 2 * H), F32),
        ],
    )(xp, W1, acat1)


def _tc_mid(accs, h, asd, W, acat, eh, scale, shift):
    return pl.pallas_call(
        _tc_mid_body,
        grid=(NB,),
        in_specs=[
            pl.BlockSpec((2, 128, ACCW), lambda i: (0, i, 0)),
            pl.BlockSpec((128, D), lambda i: (i, 0)),
            pl.BlockSpec((128, 2 * H), lambda i: (i, 0)),
            pl.BlockSpec((D, D), lambda i: (0, 0)),
            pl.BlockSpec((D, 2 * H), lambda i: (0, 0)),
            pl.BlockSpec((H, D), lambda i: (0, 0)),
            pl.BlockSpec((1, D), lambda i: (0, 0)),
            pl.BlockSpec((1, D), lambda i: (0, 0)),
        ],
        out_specs=[
            pl.BlockSpec((128, D), lambda i: (i, 0)),
            pl.BlockSpec((128, 2 * H), lambda i: (i, 0)),
        ],
        out_shape=[
            jax.ShapeDtypeStruct((NP, D), F32),
            jax.ShapeDtypeStruct((NP, 2 * H), F32),
        ],
    )(accs, h, asd, W, acat, eh, scale, shift)


def _tc_last(accs, h, asd, W3p, eh, scale, shift):
    return pl.pallas_call(
        _tc_last_body,
        grid=(NB,),
        in_specs=[
            pl.BlockSpec((2, 128, ACCW), lambda i: (0, i, 0)),
            pl.BlockSpec((128, D), lambda i: (i, 0)),
            pl.BlockSpec((128, 2 * H), lambda i: (i, 0)),
            pl.BlockSpec((D, D), lambda i: (0, 0)),
            pl.BlockSpec((H, D), lambda i: (0, 0)),
            pl.BlockSpec((1, D), lambda i: (0, 0)),
            pl.BlockSpec((1, D), lambda i: (0, 0)),
        ],
        out_specs=pl.BlockSpec((128, D), lambda i: (i, 0)),
        out_shape=jax.ShapeDtypeStruct((NP, D), F32),
    )(accs, h, asd, W3p, eh, scale, shift)


def _tc4(num0, num1, den0, den1, h2d, al2d, b3s):
    return pl.pallas_call(
        _tc4_body,
        out_shape=jax.ShapeDtypeStruct((NB, 128), F32),
    )(num0, num1, den0, den1, h2d, al2d, b3s)


def _vperm(x, idx):
    return lax.gather(
        x, idx[:, None],
        lax.GatherDimensionNumbers(offset_dims=(), collapsed_slice_dims=(0,),
                                   start_index_map=(0,)),
        (1,), mode=lax.GatherScatterMode.PROMISE_IN_BOUNDS)


def _sc_edge_body(src_hbm, dst_hbm, asd_hbm, h_hbm, zeros_hbm, out_hbm,
                  idx_s2, idx_d2, asd_s2, asd_d2, hrows2, contrib,
                  acc_sh, semg0, semg1, semi0, semi1):
    c = lax.axis_index("c")
    s = lax.axis_index("s")
    r0 = s * ROWS_PER_TILE
    pltpu.sync_copy(zeros_hbm.at[pl.ds(r0, ROWS_PER_TILE)],
                    acc_sh.at[pl.ds(r0, ROWS_PER_TILE)])
    plsc.subcore_barrier()

    nb = jnp.where(c == 0, NB_FAST, NB_SLOW)
    ebase = c * (16 * NB_FAST * EB) + s * nb * EB

    shift_idx = (lax.iota(jnp.int32, 16) + 8) & 15
    splat = [jnp.full((16,), hh, jnp.int32) for hh in range(H)]
    semg = (semg0, semg1)
    semi = (semi0, semi1)

    def stage_idx(i, sl, sem):
        return (pltpu.async_copy(src_hbm.at[pl.ds(ebase + i * EB, EB)],
                                 idx_s2.at[sl], sem),
                pltpu.async_copy(dst_hbm.at[pl.ds(ebase + i * EB, EB)],
                                 idx_d2.at[sl], sem))

    def issue(gb, sl):
        return (pltpu.async_copy(asd_hbm.at[idx_s2.at[sl]], asd_s2.at[gb],
                                 semg[gb]),
                pltpu.async_copy(asd_hbm.at[idx_d2.at[sl]], asd_d2.at[gb],
                                 semg[gb]),
                pltpu.async_copy(h_hbm.at[idx_s2.at[sl]], hrows2.at[gb],
                                 semg[gb]))

    def drain(cps):
        for cp in cps:
            cp.wait()

    for b in range(2):
        drain(stage_idx(b, b, semi[1]))
    gather_spec = [issue(0, 0), issue(1, 1)]

    def outer_body(io, carry):
        for b in range(4):
            i = io * 4 + b
            gb = b % 2
            sl = (b + 2) % 4
            drain(gather_spec[gb])
            inext = lax.rem(i + 2, nb)
            cpi = stage_idx(inext, sl, semi[0])

            @plsc.parallel_loop(0, EB, unroll=4)
            def _edges(e):
                rs = asd_s2[gb, e, :]
                rd = asd_d2[gb, e, :]
                rd8 = _vperm(rd, shift_idx)
                alpha = rs + rd8
                w = jnp.exp(jnp.maximum(alpha, 0.2 * alpha))
                contrib[e, pl.ds(D, 16)] = w
                for hh in range(H):
                    wv = _vperm(w, splat[hh])
                    contrib[e, pl.ds(hh * 16, 16)] = (
                        wv * hrows2[gb, e, pl.ds(hh * 16, 16)])
            pltpu.sync_copy(contrib, acc_sh.at[idx_d2.at[b]],
                            add=True)
            drain(cpi)
            issue(gb, sl)
        return carry

    lax.fori_loop(0, nb // 4, outer_body, 0)
    drain(gather_spec[0])
    drain(gather_spec[1])
    plsc.subcore_barrier()
    pltpu.sync_copy(acc_sh.at[pl.ds(r0, ROWS_PER_TILE)],
                    out_hbm.at[c, pl.ds(r0, ROWS_PER_TILE)])


def _sc_edge3_body(src_hbm, dst_hbm, h3_hbm, att_hbm, zeros_hbm, out_hbm,
                   idx_s, idx_d, h3_v, att_v, contrib, acc_sh, sem):
    c = lax.axis_index("c")
    s = lax.axis_index("s")
    wid = s * 2 + c
    r0 = s * ROWS_PER_TILE
    pltpu.sync_copy(zeros_hbm.at[pl.ds(r0, ROWS_PER_TILE)],
                    acc_sh.at[pl.ds(r0, ROWS_PER_TILE)])
    pltpu.sync_copy(h3_hbm, h3_v)
    pltpu.sync_copy(att_hbm, att_v)
    plsc.subcore_barrier()

    sv = att_v[0, :]
    dv = att_v[1, :]
    zcol = jnp.zeros((16,), jnp.int32)
    ocol = jnp.full((16,), 1, jnp.int32)
    zero16 = jnp.zeros((16,), F32)

    def zrow(r, carry):
        contrib[r, :] = zero16
        return carry

    lax.fori_loop(0, EB, zrow, 0)

    def batch_body(i, carry):
        base = wid * EPW3 + i * EB
        pltpu.sync_copy(src_hbm.at[pl.ds(base, EB)], idx_s)
        pltpu.sync_copy(dst_hbm.at[pl.ds(base, EB)], idx_d)
        for g in range(EB // 16):
            src16 = idx_s[pl.ds(g * 16, 16)]
            dst16 = idx_d[pl.ds(g * 16, 16)]
            hsrc = plsc.load_gather(h3_v, [src16])
            hdst = plsc.load_gather(h3_v, [dst16])
            al = sv * hsrc + dv * hdst
            w = jnp.exp(jnp.maximum(al, 0.2 * al))
            eid = lax.iota(jnp.int32, 16) + g * 16
            plsc.store_scatter(contrib, [eid, zcol], w * hsrc)
            plsc.store_scatter(contrib, [eid, ocol], w)
        pltpu.sync_copy(contrib, acc_sh.at[idx_d], add=True)
        return carry

    lax.fori_loop(0, NBATCH3, batch_body, 0)
    plsc.subcore_barrier()
    pltpu.sync_copy(acc_sh.at[pl.ds(r0, ROWS_PER_TILE)],
                    out_hbm.at[c, pl.ds(r0, ROWS_PER_TILE)])


@functools.lru_cache(maxsize=None)
def _sc_kernels():
    mesh = plsc.VectorSubcoreMesh(core_axis_name="c", subcore_axis_name="s",
                                  num_cores=2, num_subcores=16)
    params = pltpu.CompilerParams(use_tc_tiling_on_sc=False,
                                  needs_layout_passes=False)
    sc_edge = pl.kernel(
        _sc_edge_body,
        out_type=jax.ShapeDtypeStruct((2, NP, ACCW), F32),
        mesh=mesh,
        compiler_params=params,
        scratch_types=[
            pltpu.VMEM((4, EB), jnp.int32),
            pltpu.VMEM((4, EB), jnp.int32),
            pltpu.VMEM((2, EB, 16), F32),
            pltpu.VMEM((2, EB, 16), F32),
            pltpu.VMEM((2, EB, D), F32),
            pltpu.VMEM((EB, ACCW), F32),
            pltpu.VMEM_SHARED((NP, ACCW), F32),
            pltpu.SemaphoreType.DMA,
            pltpu.SemaphoreType.DMA,
            pltpu.SemaphoreType.DMA,
            pltpu.SemaphoreType.DMA,
        ],
    )
    sc_edge3 = pl.kernel(
        _sc_edge3_body,
        out_type=jax.ShapeDtypeStruct((2, NP, 16), F32),
        mesh=mesh,
        compiler_params=params,
        scratch_types=[
            pltpu.VMEM((EB,), jnp.int32),
            pltpu.VMEM((EB,), jnp.int32),
            pltpu.VMEM((NP,), F32),
            pltpu.VMEM((2, 16), F32),
            pltpu.VMEM((EB, 16), F32),
            pltpu.VMEM_SHARED((NP, 16), F32),
            pltpu.SemaphoreType.DMA,
        ],
    )
    return sc_edge, sc_edge3


def _acat(asrc, adst):
    eye = jnp.eye(H, dtype=F32)
    a_s = (asrc[:, :, None] * eye[:, None, :]).reshape(H * HD, H)
    a_d = (adst[:, :, None] * eye[:, None, :]).reshape(H * HD, H)
    return jnp.concatenate([a_s, a_d], axis=1)


def kernel(x, edge_index, W1, a_src1, a_dst1, b1, g1, be1,
           W2, a_src2, a_dst2, b2, g2, be2,
           W3, a_src3, a_dst3, b3):
    xp = jnp.zeros((NP, D), F32).at[:N_NODES].set(x)
    src = jnp.zeros((E_PAD,), jnp.int32).at[:N_EDGES].set(edge_index[0])
    dst = jnp.full((E_PAD,), NP - 1, jnp.int32).at[:N_EDGES].set(
        edge_index[1])

    acat1 = _acat(a_src1, a_dst1)
    acat2 = _acat(a_src2, a_dst2)
    eh = (jnp.eye(H, dtype=F32)[:, :, None]
          * jnp.ones((HD,), F32)).reshape(H, H * HD)
    inv = 1.0 / jnp.sqrt(jnp.float32(1.0 + 1e-5))
    sc1 = (g1 * inv).reshape(1, D)
    sh1 = (b1 * g1 * inv + be1).reshape(1, D)
    sc2 = (g2 * inv).reshape(1, D)
    sh2 = (b2 * g2 * inv + be2).reshape(1, D)

    s3 = a_src3[0, 0]
    d3 = a_dst3[0, 0]
    W3p = jnp.zeros((D, D), F32)
    W3p = W3p.at[:, 0].set(W3[:, 0])
    W3p = W3p.at[:, 1].set(W3[:, 0] * (s3 + d3))
    att3 = jnp.stack([jnp.full((16,), s3, F32), jnp.full((16,), d3, F32)])

    zeros_acc = jnp.zeros((NP, ACCW), F32)
    zeros_a3 = jnp.zeros((NP, 16), F32)
    _sc_edge, _sc_edge3 = _sc_kernels()

    h1, asd1 = _tc1(xp, W1, acat1)
    accs1 = _sc_edge(src, dst, asd1, h1, zeros_acc)
    h2, asd2 = _tc_mid(accs1, h1, asd1, W2, acat2, eh, sc1, sh1)
    accs2 = _sc_edge(src, dst, asd2, h2, zeros_acc)
    h3full = _tc_last(accs2, h2, asd2, W3p, eh, sc2, sh2)

    h3 = h3full[:, 0]
    al3 = h3full[:, 1]
    accs3 = _sc_edge3(src, dst, h3, att3, zeros_a3)

    num0 = accs3[0, :, 0].reshape(NB, 128)
    num1 = accs3[1, :, 0].reshape(NB, 128)
    den0 = accs3[0, :, 1].reshape(NB, 128)
    den1 = accs3[1, :, 1].reshape(NB, 128)
    h2d = h3.reshape(NB, 128)
    al2d = al3.reshape(NB, 128)
    b3s = jnp.broadcast_to(b3.reshape(1, 1), (1, 128)).astype(F32)

    out2d = _tc4(num0, num1, den0, den1, h2d, al2d, b3s)
    return out2d.reshape(NP, 1)[:N_NODES]
